# Initial kernel scaffold; baseline (speedup 1.0000x reference)
import jax
import jax.numpy as jnp
from jax.experimental import pallas as pl


def kernel(x, W, b, assign):
    raise NotImplementedError("write your pallas kernel here")


# speedup vs baseline: 1.9846x; 1.9846x over previous
import functools

import jax
import jax.numpy as jnp
from jax import lax
from jax.experimental import pallas as pl
from jax.experimental.pallas import tpu as pltpu
from jax.experimental.pallas import tpu_sc as plsc

E = 8
N = 8192
D = 1024

NC = 2
NS = 16
NW = NC * NS
CPW = N // NW
LANES = 16

T = 256
NUM_TILES = N // T
MAX_ITEMS = NUM_TILES + E

def _wid():
    return lax.axis_index("s") * NC + lax.axis_index("c")


@functools.cache
def _mesh():
    return plsc.VectorSubcoreMesh(core_axis_name="c", subcore_axis_name="s")


def _sc_params():
    return pltpu.CompilerParams(needs_layout_passes=False)


def _sc_count_body(assign_hbm, cnts_hbm, asg_v, cnt_v):
    w = _wid()
    base = w * CPW
    pltpu.sync_copy(assign_hbm.at[pl.ds(base, CPW)], asg_v)
    lane = lax.iota(jnp.int32, LANES)
    cnt = jnp.zeros((LANES,), jnp.int32)
    for j in range(CPW // LANES):
        vals = asg_v[pl.ds(j * LANES, LANES)]
        for e in range(E):
            pc = jnp.sum((vals == e).astype(jnp.int32))
            cnt = cnt + jnp.where(lane == e, pc, 0)
    cnt_v[...] = cnt
    pltpu.sync_copy(cnt_v, cnts_hbm.at[w])


@functools.cache
def _sc_count():
    return pl.kernel(
        _sc_count_body,
        out_type=jax.ShapeDtypeStruct((NW, LANES), jnp.int32),
        mesh=_mesh(),
        scratch_types=[
            pltpu.VMEM((CPW,), jnp.int32),
            pltpu.VMEM((LANES,), jnp.int32),
        ],
        compiler_params=_sc_params(),
    )


def _sc_dispatch_body(x_hbm, assign_hbm, cnts_hbm, xs_hbm, dest_hbm,
                      asg_v, cnt_v, dest_v, xbuf_v, sem):
    w = _wid()
    base = w * CPW
    pltpu.sync_copy(assign_hbm.at[pl.ds(base, CPW)], asg_v)
    pltpu.sync_copy(cnts_hbm, cnt_v)

    lane = lax.iota(jnp.int32, LANES)
    tot = jnp.zeros((LANES,), jnp.int32)
    pre = jnp.zeros((LANES,), jnp.int32)
    for wp in range(NW):
        row = cnt_v[wp]
        tot = tot + row
        pred = jnp.full((LANES,), wp, jnp.int32) < w
        pre = pre + jnp.where(pred, row, 0)
    off = plsc.cumsum(tot) - tot
    cur = off + pre

    for j in range(CPW // LANES):
        vals = asg_v[pl.ds(j * LANES, LANES)]
        dest = jnp.zeros((LANES,), jnp.int32)
        for e in range(E):
            m = vals == e
            mi = m.astype(jnp.int32)
            cs = plsc.cumsum(mi)
            cur_e = jnp.sum(jnp.where(lane == e, cur, 0))
            tot_e = jnp.sum(mi)
            dest = jnp.where(m, cur_e + cs - 1, dest)
            cur = cur + jnp.where(lane == e, tot_e, 0)
        dest_v[j // 4, pl.ds((j % 4) * LANES, LANES)] = dest

    pltpu.sync_copy(dest_v, dest_hbm.at[w])

    for j in range(CPW // LANES):
        dv = dest_v[j // 4, pl.ds((j % 4) * LANES, LANES)]
        pltpu.sync_copy(x_hbm.at[pl.ds(base + j * LANES, LANES)], xbuf_v)
        pltpu.async_copy(xbuf_v, xs_hbm.at[dv], sem).wait()


@functools.cache
def _sc_dispatch():
    return pl.kernel(
        _sc_dispatch_body,
        out_type=(
            jax.ShapeDtypeStruct((N, D), jnp.float32),
            jax.ShapeDtypeStruct((NW, CPW // 64, 64), jnp.int32),
        ),
        mesh=_mesh(),
        scratch_types=[
            pltpu.VMEM((CPW,), jnp.int32),
            pltpu.VMEM((NW, LANES), jnp.int32),
            pltpu.VMEM((CPW // 64, 64), jnp.int32),
            pltpu.VMEM((LANES, D), jnp.float32),
            pltpu.SemaphoreType.DMA,
        ],
        compiler_params=_sc_params(),
    )


def _sc_gather_body(ys_hbm, dest_hbm, out_hbm, dest_v, buf_v, sem):
    w = _wid()
    base = w * CPW
    pltpu.sync_copy(dest_hbm.at[w], dest_v)
    for j in range(CPW // 64):
        pltpu.async_copy(ys_hbm.at[dest_v.at[j]], buf_v, sem).wait()
        pltpu.sync_copy(buf_v, out_hbm.at[pl.ds(base + j * 64, 64)])


@functools.cache
def _sc_gather():
    return pl.kernel(
        _sc_gather_body,
        out_type=jax.ShapeDtypeStruct((N, D), jnp.float32),
        mesh=_mesh(),
        scratch_types=[
            pltpu.VMEM((CPW // 64, 64), jnp.int32),
            pltpu.VMEM((64, D), jnp.float32),
            pltpu.SemaphoreType.DMA,
        ],
        compiler_params=_sc_params(),
    )


def _gmm_body(it_tile, it_e, it_start, it_end, it_valid,
              x_ref, w_ref, b_ref, out_ref):
    i = pl.program_id(0)
    start = it_start[i]
    end = it_end[i]
    tile = it_tile[i]

    @pl.when(it_valid[i] == 1)
    def _():
        rows = tile * T + lax.broadcasted_iota(jnp.int32, (T, 1), 0)
        m = (rows >= start) & (rows < end)
        y = jnp.dot(x_ref[...], w_ref[0],
                    preferred_element_type=jnp.float32) + b_ref[0]
        out_ref[...] = jnp.where(m, y, out_ref[...])


def _gmm(it_tile, it_e, it_start, it_end, it_valid, xs, W, b):
    grid_spec = pltpu.PrefetchScalarGridSpec(
        num_scalar_prefetch=5,
        grid=(MAX_ITEMS,),
        in_specs=[
            pl.BlockSpec((T, D), lambda i, tl, ex, st, en, va: (tl[i], 0)),
            pl.BlockSpec((1, D, D), lambda i, tl, ex, st, en, va: (ex[i], 0, 0)),
            pl.BlockSpec((1, 1, D), lambda i, tl, ex, st, en, va: (ex[i], 0, 0)),
        ],
        out_specs=pl.BlockSpec((T, D), lambda i, tl, ex, st, en, va: (tl[i], 0)),
    )
    return pl.pallas_call(
        _gmm_body,
        grid_spec=grid_spec,
        out_shape=jax.ShapeDtypeStruct((N, D), jnp.float32),
    )(it_tile, it_e, it_start, it_end, it_valid, xs, W,
      b.reshape(E, 1, D))


def kernel(x, W, b, assign):
    assign = assign.astype(jnp.int32)
    cnts = _sc_count()(assign)
    xs, dest = _sc_dispatch()(x, assign, cnts)

    totals = jnp.sum(cnts, axis=0)[:E]
    ends = jnp.cumsum(totals)
    starts = ends - totals
    first_tile = starts // T
    last_tile_ex = (ends + T - 1) // T
    ntiles = jnp.where(totals > 0, last_tile_ex - first_tile, 0)
    csum = jnp.cumsum(ntiles)
    total_items = csum[-1]
    i = jnp.arange(MAX_ITEMS, dtype=jnp.int32)
    ic = jnp.minimum(i, total_items - 1)
    e_of = jnp.searchsorted(csum, ic, side="right").astype(jnp.int32)
    it_tile = (first_tile[e_of] + (ic - (csum[e_of] - ntiles[e_of]))).astype(jnp.int32)
    it_e = e_of
    it_start = starts[e_of].astype(jnp.int32)
    it_end = ends[e_of].astype(jnp.int32)
    it_valid = (i < total_items).astype(jnp.int32)

    ys = _gmm(it_tile, it_e, it_start, it_end, it_valid, xs, W, b)
    return _sc_gather()(ys, dest)
